# Initial kernel scaffold; baseline (speedup 1.0000x reference)
import jax
import jax.numpy as jnp
from jax.experimental import pallas as pl


def kernel(x, adj_mat, W1, a1, W2, a2):
    raise NotImplementedError("write your pallas kernel here")


# speedup vs baseline: 1.5105x; 1.5105x over previous
import functools

import jax
import jax.numpy as jnp
from jax.experimental import pallas as pl

_NEG_SLOPE = 0.2


def _proj_body(x_ref, w_ref, g_ref):
    g_ref[...] = jnp.dot(x_ref[...], w_ref[...], preferred_element_type=jnp.float32)


def _attend(el, er, adj, g_h):
    e = el[:, None] + er[None, :]
    e = jnp.where(e >= 0.0, e, _NEG_SLOPE * e)
    em = jnp.where(adj, e, -jnp.inf)
    m = jnp.max(em, axis=1, keepdims=True)
    p = jnp.exp(em - m)
    s = jnp.sum(p, axis=1, keepdims=True)
    o = jnp.dot(p, g_h, preferred_element_type=jnp.float32)
    return o / s


def _layer1_body(g_blk_ref, g_all_ref, adj_ref, a1_ref, w2_ref, g2_ref,
                 *, n_heads, f1):
    g_blk = g_blk_ref[...]
    adj = adj_ref[...]
    acc = None
    for h in range(n_heads):
        sl = slice(h * f1, (h + 1) * f1)
        a_l = a1_ref[h, :f1]
        a_r = a1_ref[h, f1:]
        g_h = g_all_ref[:, sl]
        el = jnp.sum(g_blk[:, sl] * a_l[None, :], axis=1)
        er = jnp.sum(g_h * a_r[None, :], axis=1)
        o = _attend(el, er, adj, g_h)
        acc = o if acc is None else acc + o
    hmean = acc * (1.0 / n_heads)
    he = jnp.where(hmean > 0.0, hmean, jnp.exp(hmean) - 1.0)
    g2_ref[...] = jnp.dot(he, w2_ref[...], preferred_element_type=jnp.float32)


def _layer2_body(g_blk_ref, g_all_ref, adj_ref, a2_ref, out_ref, *, c):
    g_blk = g_blk_ref[...]
    g_all = g_all_ref[...]
    a_l = a2_ref[0, :c]
    a_r = a2_ref[0, c:]
    el = jnp.sum(g_blk * a_l[None, :], axis=1)
    er = jnp.sum(g_all * a_r[None, :], axis=1)
    out_ref[...] = _attend(el, er, adj_ref[...], g_all)


def kernel(x, adj_mat, W1, a1, W2, a2):
    n, fin = x.shape
    htot = W1.shape[1]
    n_heads = a1.shape[0]
    f1 = htot // n_heads
    c = W2.shape[1]
    adj = adj_mat.reshape(n, n)
    bm = 256 if n % 256 == 0 else n

    g1 = pl.pallas_call(
        _proj_body,
        grid=(n // bm,),
        in_specs=[
            pl.BlockSpec((bm, fin), lambda i: (i, 0)),
            pl.BlockSpec((fin, htot), lambda i: (0, 0)),
        ],
        out_specs=pl.BlockSpec((bm, htot), lambda i: (i, 0)),
        out_shape=jax.ShapeDtypeStruct((n, htot), jnp.float32),
    )(x, W1)

    g2 = pl.pallas_call(
        functools.partial(_layer1_body, n_heads=n_heads, f1=f1),
        grid=(n // bm,),
        in_specs=[
            pl.BlockSpec((bm, htot), lambda i: (i, 0)),
            pl.BlockSpec((n, htot), lambda i: (0, 0)),
            pl.BlockSpec((bm, n), lambda i: (i, 0)),
            pl.BlockSpec((n_heads, 2 * f1), lambda i: (0, 0)),
            pl.BlockSpec((f1, c), lambda i: (0, 0)),
        ],
        out_specs=pl.BlockSpec((bm, c), lambda i: (i, 0)),
        out_shape=jax.ShapeDtypeStruct((n, c), jnp.float32),
    )(g1, g1, adj, a1, W2)

    out = pl.pallas_call(
        functools.partial(_layer2_body, c=c),
        grid=(n // bm,),
        in_specs=[
            pl.BlockSpec((bm, c), lambda i: (i, 0)),
            pl.BlockSpec((n, c), lambda i: (0, 0)),
            pl.BlockSpec((bm, n), lambda i: (i, 0)),
            pl.BlockSpec((1, 2 * c), lambda i: (0, 0)),
        ],
        out_specs=pl.BlockSpec((bm, c), lambda i: (i, 0)),
        out_shape=jax.ShapeDtypeStruct((n, c), jnp.float32),
    )(g2, g2, adj, a2)

    return out


# speedup vs baseline: 2.5950x; 1.7179x over previous
import functools

import jax
import jax.numpy as jnp
from jax.experimental import pallas as pl

_NEG_SLOPE = 0.2


def _proj_body(x_ref, w_ref, a_ref, g_ref, el_ref, er_ref, *, n_heads, f1):
    g = jnp.dot(x_ref[...], w_ref[...], preferred_element_type=jnp.float32)
    g_ref[...] = g
    for h in range(n_heads):
        g_h = g[:, h * f1:(h + 1) * f1]
        el_ref[h, :] = jnp.sum(g_h * a_ref[h, :f1][None, :], axis=1)
        er_ref[h, :] = jnp.sum(g_h * a_ref[h, f1:][None, :], axis=1)


def _attend(el, er, adj, gext_h, f):
    mel = jnp.max(el)
    mer = jnp.max(er)
    e1l = jnp.exp(el - mel).astype(jnp.bfloat16)
    e2l = jnp.exp(_NEG_SLOPE * el - mel).astype(jnp.bfloat16)
    e1r = jnp.exp(er - mer).astype(jnp.bfloat16)
    e2r = jnp.exp(_NEG_SLOPE * er - mer).astype(jnp.bfloat16)
    t = jnp.maximum(e1l[:, None] * e1r[None, :], e2l[:, None] * e2r[None, :])
    p = jnp.where(adj, t, jnp.bfloat16(0.0))
    oe = jnp.dot(p, gext_h, preferred_element_type=jnp.float32)
    return oe[:, :f] / oe[:, f:f + 1]


def _layer1_body(el_ref, er_ref, adj_ref, gext_ref, w2_ref, g2_ref,
                 *, n_heads, f1):
    adj = adj_ref[...]
    acc = None
    for h in range(n_heads):
        o = _attend(el_ref[h, :], er_ref[h, :], adj,
                    gext_ref[:, h * 128:(h + 1) * 128], f1)
        acc = o if acc is None else acc + o
    hmean = acc * (1.0 / n_heads)
    he = jnp.where(hmean > 0.0, hmean, jnp.exp(hmean) - 1.0)
    g2_ref[...] = jnp.dot(he, w2_ref[...], preferred_element_type=jnp.float32)


def _layer2_body(g2_blk_ref, g2_all_ref, adj_ref, g2ext_ref, a2_ref, out_ref,
                 *, c):
    a_l = a2_ref[0, :c]
    a_r = a2_ref[0, c:]
    el = jnp.sum(g2_blk_ref[...] * a_l[None, :], axis=1)
    er = jnp.sum(g2_all_ref[...] * a_r[None, :], axis=1)
    out_ref[...] = _attend(el, er, adj_ref[...], g2ext_ref[...], c)


def _ones_augment(g, f, width):
    n = g.shape[0]
    n_heads = g.shape[1] // f
    parts = []
    for h in range(n_heads):
        parts.append(g[:, h * f:(h + 1) * f].astype(jnp.bfloat16))
        parts.append(jnp.ones((n, 1), jnp.bfloat16))
        parts.append(jnp.zeros((n, width - f - 1), jnp.bfloat16))
    return jnp.concatenate(parts, axis=1)


def kernel(x, adj_mat, W1, a1, W2, a2):
    n, fin = x.shape
    htot = W1.shape[1]
    n_heads = a1.shape[0]
    f1 = htot // n_heads
    c = W2.shape[1]
    adj = adj_mat.reshape(n, n)
    bm = 256 if n % 256 == 0 else n
    grid = (n // bm,)

    g1, el1, er1 = pl.pallas_call(
        functools.partial(_proj_body, n_heads=n_heads, f1=f1),
        grid=grid,
        in_specs=[
            pl.BlockSpec((bm, fin), lambda i: (i, 0)),
            pl.BlockSpec((fin, htot), lambda i: (0, 0)),
            pl.BlockSpec((n_heads, 2 * f1), lambda i: (0, 0)),
        ],
        out_specs=[
            pl.BlockSpec((bm, htot), lambda i: (i, 0)),
            pl.BlockSpec((n_heads, bm), lambda i: (0, i)),
            pl.BlockSpec((n_heads, bm), lambda i: (0, i)),
        ],
        out_shape=[
            jax.ShapeDtypeStruct((n, htot), jnp.float32),
            jax.ShapeDtypeStruct((n_heads, n), jnp.float32),
            jax.ShapeDtypeStruct((n_heads, n), jnp.float32),
        ],
    )(x, W1, a1)

    g1ext = _ones_augment(g1, f1, 128)

    g2 = pl.pallas_call(
        functools.partial(_layer1_body, n_heads=n_heads, f1=f1),
        grid=grid,
        in_specs=[
            pl.BlockSpec((n_heads, bm), lambda i: (0, i)),
            pl.BlockSpec((n_heads, n), lambda i: (0, 0)),
            pl.BlockSpec((bm, n), lambda i: (i, 0)),
            pl.BlockSpec((n, n_heads * 128), lambda i: (0, 0)),
            pl.BlockSpec((f1, c), lambda i: (0, 0)),
        ],
        out_specs=pl.BlockSpec((bm, c), lambda i: (i, 0)),
        out_shape=jax.ShapeDtypeStruct((n, c), jnp.float32),
    )(el1, er1, adj, g1ext, W2)

    g2ext = _ones_augment(g2, c, 128)

    out = pl.pallas_call(
        functools.partial(_layer2_body, c=c),
        grid=grid,
        in_specs=[
            pl.BlockSpec((bm, c), lambda i: (i, 0)),
            pl.BlockSpec((n, c), lambda i: (0, 0)),
            pl.BlockSpec((bm, n), lambda i: (i, 0)),
            pl.BlockSpec((n, 128), lambda i: (0, 0)),
            pl.BlockSpec((1, 2 * c), lambda i: (0, 0)),
        ],
        out_specs=pl.BlockSpec((bm, c), lambda i: (i, 0)),
        out_shape=jax.ShapeDtypeStruct((n, c), jnp.float32),
    )(g2, g2, adj, g2ext, a2)

    return out


# speedup vs baseline: 3.0868x; 1.1895x over previous
import functools

import jax
import jax.numpy as jnp
from jax.experimental import pallas as pl

_NEG_SLOPE = 0.2


def _proj_body(x_ref, w_ref, a_ref, g_ref, el_ref, er_ref, *, n_heads, f1):
    g = jnp.dot(x_ref[...], w_ref[...], preferred_element_type=jnp.float32)
    g_ref[...] = g
    for h in range(n_heads):
        g_h = g[:, h * f1:(h + 1) * f1]
        el_ref[h, :] = jnp.sum(g_h * a_ref[h, :f1][None, :], axis=1)
        er_ref[h, :] = jnp.sum(g_h * a_ref[h, f1:][None, :], axis=1)


def _attend(el, er, adj, gext_h, f):
    mel = jnp.max(el)
    mer = jnp.max(er)
    e1l = jnp.exp(el - mel).astype(jnp.bfloat16)
    e2l = jnp.exp(_NEG_SLOPE * el - mel).astype(jnp.bfloat16)
    e1r = jnp.exp(er - mer).astype(jnp.bfloat16)
    e2r = jnp.exp(_NEG_SLOPE * er - mer).astype(jnp.bfloat16)
    t = jnp.maximum(e1l[:, None] * e1r[None, :], e2l[:, None] * e2r[None, :])
    p = jnp.where(adj, t, jnp.bfloat16(0.0))
    oe = jnp.dot(p, gext_h, preferred_element_type=jnp.float32)
    return oe[:, :f] / oe[:, f:f + 1]


def _layer1_body(el_ref, er_ref, adj_ref, gext_ref, w2_ref, g2_ref,
                 *, n_heads, f1):
    adj = adj_ref[...]
    acc = None
    for h in range(n_heads):
        o = _attend(el_ref[h, :], er_ref[h, :], adj,
                    gext_ref[:, h * 128:(h + 1) * 128], f1)
        acc = o if acc is None else acc + o
    hmean = acc * (1.0 / n_heads)
    he = jnp.where(hmean > 0.0, hmean, jnp.exp(hmean) - 1.0)
    g2_ref[...] = jnp.dot(he, w2_ref[...], preferred_element_type=jnp.float32)


def _layer2_body(g2_blk_ref, g2_all_ref, adj_ref, g2ext_ref, a2_ref, out_ref,
                 *, c):
    a_l = a2_ref[0, :c]
    a_r = a2_ref[0, c:]
    el = jnp.sum(g2_blk_ref[...] * a_l[None, :], axis=1)
    er = jnp.sum(g2_all_ref[...] * a_r[None, :], axis=1)
    out_ref[...] = _attend(el, er, adj_ref[...], g2ext_ref[...], c)


def _ones_augment(g, f, width):
    n = g.shape[0]
    n_heads = g.shape[1] // f
    parts = []
    for h in range(n_heads):
        parts.append(g[:, h * f:(h + 1) * f].astype(jnp.bfloat16))
        parts.append(jnp.ones((n, 1), jnp.bfloat16))
        parts.append(jnp.zeros((n, width - f - 1), jnp.bfloat16))
    return jnp.concatenate(parts, axis=1)


def kernel(x, adj_mat, W1, a1, W2, a2):
    n, fin = x.shape
    htot = W1.shape[1]
    n_heads = a1.shape[0]
    f1 = htot // n_heads
    c = W2.shape[1]
    adj = adj_mat.reshape(n, n)
    bm = 512 if n % 512 == 0 else n
    grid = (n // bm,)

    g1, el1, er1 = pl.pallas_call(
        functools.partial(_proj_body, n_heads=n_heads, f1=f1),
        grid=grid,
        in_specs=[
            pl.BlockSpec((bm, fin), lambda i: (i, 0)),
            pl.BlockSpec((fin, htot), lambda i: (0, 0)),
            pl.BlockSpec((n_heads, 2 * f1), lambda i: (0, 0)),
        ],
        out_specs=[
            pl.BlockSpec((bm, htot), lambda i: (i, 0)),
            pl.BlockSpec((n_heads, bm), lambda i: (0, i)),
            pl.BlockSpec((n_heads, bm), lambda i: (0, i)),
        ],
        out_shape=[
            jax.ShapeDtypeStruct((n, htot), jnp.float32),
            jax.ShapeDtypeStruct((n_heads, n), jnp.float32),
            jax.ShapeDtypeStruct((n_heads, n), jnp.float32),
        ],
    )(x, W1, a1)

    g1ext = _ones_augment(g1, f1, 128)

    g2 = pl.pallas_call(
        functools.partial(_layer1_body, n_heads=n_heads, f1=f1),
        grid=grid,
        in_specs=[
            pl.BlockSpec((n_heads, bm), lambda i: (0, i)),
            pl.BlockSpec((n_heads, n), lambda i: (0, 0)),
            pl.BlockSpec((bm, n), lambda i: (i, 0)),
            pl.BlockSpec((n, n_heads * 128), lambda i: (0, 0)),
            pl.BlockSpec((f1, c), lambda i: (0, 0)),
        ],
        out_specs=pl.BlockSpec((bm, c), lambda i: (i, 0)),
        out_shape=jax.ShapeDtypeStruct((n, c), jnp.float32),
    )(el1, er1, adj, g1ext, W2)

    g2ext = _ones_augment(g2, c, 128)

    out = pl.pallas_call(
        functools.partial(_layer2_body, c=c),
        grid=grid,
        in_specs=[
            pl.BlockSpec((bm, c), lambda i: (i, 0)),
            pl.BlockSpec((n, c), lambda i: (0, 0)),
            pl.BlockSpec((bm, n), lambda i: (i, 0)),
            pl.BlockSpec((n, 128), lambda i: (0, 0)),
            pl.BlockSpec((1, 2 * c), lambda i: (0, 0)),
        ],
        out_specs=pl.BlockSpec((bm, c), lambda i: (i, 0)),
        out_shape=jax.ShapeDtypeStruct((n, c), jnp.float32),
    )(g2, g2, adj, g2ext, a2)

    return out


# speedup vs baseline: 3.4569x; 1.1199x over previous
import functools

import jax
import jax.numpy as jnp
from jax.experimental import pallas as pl

_NEG_SLOPE = 0.2


def _augment(g, f, n_heads):
    bm = g.shape[0]
    parts = []
    for h in range(n_heads):
        parts.append(g[:, h * f:(h + 1) * f].astype(jnp.bfloat16))
        parts.append(jnp.ones((bm, 1), jnp.bfloat16))
        parts.append(jnp.zeros((bm, 128 - f - 1), jnp.bfloat16))
    return jnp.concatenate(parts, axis=1)


def _proj_body(x_ref, w_ref, a_ref, gext_ref, el_ref, er_ref, *, n_heads, f1):
    g = jnp.dot(x_ref[...], w_ref[...], preferred_element_type=jnp.float32)
    for h in range(n_heads):
        g_h = g[:, h * f1:(h + 1) * f1]
        el_ref[h, :] = jnp.sum(g_h * a_ref[h, :f1][None, :], axis=1)
        er_ref[h, :] = jnp.sum(g_h * a_ref[h, f1:][None, :], axis=1)
    gext_ref[...] = _augment(g, f1, n_heads)


def _attend(el, er, adj, gext_h, f):
    mel = jnp.max(el)
    mer = jnp.max(er)
    e1l = jnp.exp(el - mel).astype(jnp.bfloat16)
    e2l = jnp.exp(_NEG_SLOPE * el - mel).astype(jnp.bfloat16)
    e1r = jnp.exp(er - mer).astype(jnp.bfloat16)
    e2r = jnp.exp(_NEG_SLOPE * er - mer).astype(jnp.bfloat16)
    t = jnp.maximum(e1l[:, None] * e1r[None, :], e2l[:, None] * e2r[None, :])
    p = jnp.where(adj, t, jnp.bfloat16(0.0))
    oe = jnp.dot(p, gext_h, preferred_element_type=jnp.float32)
    return oe[:, :f] / oe[:, f:f + 1]


def _layer1_body(el_ref, er_ref, adj_ref, gext_ref, w2_ref, g2_ref, g2ext_ref,
                 *, n_heads, f1):
    adj = adj_ref[...]
    acc = None
    for h in range(n_heads):
        o = _attend(el_ref[h, :], er_ref[h, :], adj,
                    gext_ref[:, h * 128:(h + 1) * 128], f1)
        acc = o if acc is None else acc + o
    hmean = acc * (1.0 / n_heads)
    he = jnp.where(hmean > 0.0, hmean, jnp.exp(hmean) - 1.0)
    g2 = jnp.dot(he, w2_ref[...], preferred_element_type=jnp.float32)
    g2_ref[...] = g2
    g2ext_ref[...] = _augment(g2, g2.shape[1], 1)


def _layer2_body(g2_blk_ref, g2_all_ref, adj_ref, g2ext_ref, a2_ref, out_ref,
                 *, c):
    a_l = a2_ref[0, :c]
    a_r = a2_ref[0, c:]
    el = jnp.sum(g2_blk_ref[...] * a_l[None, :], axis=1)
    er = jnp.sum(g2_all_ref[...] * a_r[None, :], axis=1)
    out_ref[...] = _attend(el, er, adj_ref[...], g2ext_ref[...], c)


def kernel(x, adj_mat, W1, a1, W2, a2):
    n, fin = x.shape
    htot = W1.shape[1]
    n_heads = a1.shape[0]
    f1 = htot // n_heads
    c = W2.shape[1]
    adj = adj_mat.reshape(n, n)
    bm = 512 if n % 512 == 0 else n
    grid = (n // bm,)

    gext1, el1, er1 = pl.pallas_call(
        functools.partial(_proj_body, n_heads=n_heads, f1=f1),
        grid=grid,
        in_specs=[
            pl.BlockSpec((bm, fin), lambda i: (i, 0)),
            pl.BlockSpec((fin, htot), lambda i: (0, 0)),
            pl.BlockSpec((n_heads, 2 * f1), lambda i: (0, 0)),
        ],
        out_specs=[
            pl.BlockSpec((bm, n_heads * 128), lambda i: (i, 0)),
            pl.BlockSpec((n_heads, bm), lambda i: (0, i)),
            pl.BlockSpec((n_heads, bm), lambda i: (0, i)),
        ],
        out_shape=[
            jax.ShapeDtypeStruct((n, n_heads * 128), jnp.bfloat16),
            jax.ShapeDtypeStruct((n_heads, n), jnp.float32),
            jax.ShapeDtypeStruct((n_heads, n), jnp.float32),
        ],
    )(x.astype(jnp.bfloat16), W1.astype(jnp.bfloat16), a1)

    g2, g2ext = pl.pallas_call(
        functools.partial(_layer1_body, n_heads=n_heads, f1=f1),
        grid=grid,
        in_specs=[
            pl.BlockSpec((n_heads, bm), lambda i: (0, i)),
            pl.BlockSpec((n_heads, n), lambda i: (0, 0)),
            pl.BlockSpec((bm, n), lambda i: (i, 0)),
            pl.BlockSpec((n, n_heads * 128), lambda i: (0, 0)),
            pl.BlockSpec((f1, c), lambda i: (0, 0)),
        ],
        out_specs=[
            pl.BlockSpec((bm, c), lambda i: (i, 0)),
            pl.BlockSpec((bm, 128), lambda i: (i, 0)),
        ],
        out_shape=[
            jax.ShapeDtypeStruct((n, c), jnp.float32),
            jax.ShapeDtypeStruct((n, 128), jnp.bfloat16),
        ],
    )(el1, er1, adj, gext1, W2)

    out = pl.pallas_call(
        functools.partial(_layer2_body, c=c),
        grid=grid,
        in_specs=[
            pl.BlockSpec((bm, c), lambda i: (i, 0)),
            pl.BlockSpec((n, c), lambda i: (0, 0)),
            pl.BlockSpec((bm, n), lambda i: (i, 0)),
            pl.BlockSpec((n, 128), lambda i: (0, 0)),
            pl.BlockSpec((1, 2 * c), lambda i: (0, 0)),
        ],
        out_specs=pl.BlockSpec((bm, c), lambda i: (i, 0)),
        out_shape=jax.ShapeDtypeStruct((n, c), jnp.float32),
    )(g2, g2, adj, g2ext, a2)

    return out
